# Initial kernel scaffold; baseline (speedup 1.0000x reference)
import jax
import jax.numpy as jnp
from jax.experimental import pallas as pl


def kernel(feat, edge_index, weight):
    raise NotImplementedError("write your pallas kernel here")


# speedup vs baseline: 3.4962x; 3.4962x over previous
import functools

import jax
import jax.numpy as jnp
from jax import lax
from jax.experimental import pallas as pl
from jax.experimental.pallas import tpu as pltpu
from jax.experimental.pallas import tpu_sc as plsc

N = 10000
E = 320000
D = 128

NC = 2
NS = 16
NW = NC * NS
CH = 128
NCH = (E + NW * CH - 1) // (NW * CH)
EPW = CH * NCH
EPAD = EPW * NW
ACC_R = 10240
ZR = 32
RPT = ACC_R // NS
BR = 1000


def _mesh():
    return plsc.VectorSubcoreMesh(core_axis_name="c", subcore_axis_name="s")


@functools.partial(
    pl.kernel,
    mesh=_mesh(),
    out_type=jax.ShapeDtypeStruct((NC, ACC_R, D), jnp.float32),
    scratch_types=[
        pltpu.VMEM((NCH, CH), jnp.int32),
        pltpu.VMEM((CH, D), jnp.float32),
        pltpu.VMEM((ZR, D), jnp.float32),
        pltpu.VMEM_SHARED((ACC_R, D), jnp.float32),
    ],
)
def _deg_sc(e_hbm, out_hbm, didx, ones, zbuf, acc):
    c = lax.axis_index("c")
    s = lax.axis_index("s")
    wid = s * NC + c
    base = wid * EPW

    def ld(j, carry):
        pltpu.sync_copy(e_hbm.at[1, pl.ds(base + j * CH, CH)], didx.at[j])
        return carry

    lax.fori_loop(0, NCH, ld, 0)

    zv = jnp.zeros((16,), jnp.float32)
    ov = jnp.ones((16,), jnp.float32)

    def fill_ones(i, carry):
        for t in range(D // 16):
            ones[i, pl.ds(t * 16, 16)] = ov
        return carry

    lax.fori_loop(0, CH, fill_ones, 0)

    def fill_z(i, carry):
        for t in range(D // 16):
            zbuf[i, pl.ds(t * 16, 16)] = zv
        return carry

    lax.fori_loop(0, ZR, fill_z, 0)

    for b in range(RPT // ZR):
        pltpu.sync_copy(zbuf, acc.at[pl.ds(s * RPT + b * ZR, ZR)])
    plsc.subcore_barrier()

    def step(j, carry):
        pltpu.sync_copy(ones, acc.at[didx.at[j]], add=True)
        return carry

    lax.fori_loop(0, NCH, step, 0)
    plsc.subcore_barrier()

    pltpu.sync_copy(acc.at[pl.ds(s * RPT, RPT)], out_hbm.at[c, pl.ds(s * RPT, RPT)])


@functools.partial(
    pl.kernel,
    mesh=_mesh(),
    out_type=jax.ShapeDtypeStruct((NC, ACC_R, D), jnp.float32),
    scratch_types=[
        pltpu.VMEM((NCH, CH), jnp.int32),
        pltpu.VMEM((NCH, CH), jnp.int32),
        pltpu.VMEM((CH, D), jnp.float32),
        pltpu.VMEM((ZR, D), jnp.float32),
        pltpu.VMEM_SHARED((ACC_R, D), jnp.float32),
        pltpu.SemaphoreType.DMA,
    ],
)
def _prop_sc(hs_hbm, e_hbm, out_hbm, sidx, didx, rows, zbuf, acc, sem):
    c = lax.axis_index("c")
    s = lax.axis_index("s")
    wid = s * NC + c
    base = wid * EPW

    def ld(j, carry):
        pltpu.sync_copy(e_hbm.at[0, pl.ds(base + j * CH, CH)], sidx.at[j])
        pltpu.sync_copy(e_hbm.at[1, pl.ds(base + j * CH, CH)], didx.at[j])
        return carry

    lax.fori_loop(0, NCH, ld, 0)

    zv = jnp.zeros((16,), jnp.float32)

    def fill(i, carry):
        for t in range(D // 16):
            zbuf[i, pl.ds(t * 16, 16)] = zv
        return carry

    lax.fori_loop(0, ZR, fill, 0)

    per_tile = ACC_R // NS
    for b in range(per_tile // ZR):
        pltpu.sync_copy(zbuf, acc.at[pl.ds(s * per_tile + b * ZR, ZR)])
    plsc.subcore_barrier()

    def step(j, carry):
        pltpu.async_copy(hs_hbm.at[sidx.at[j]], rows, sem).wait()
        pltpu.sync_copy(rows, acc.at[didx.at[j]], add=True)
        return carry

    lax.fori_loop(0, NCH, step, 0)
    plsc.subcore_barrier()

    pltpu.sync_copy(acc.at[pl.ds(s * RPT, RPT)], out_hbm.at[c, pl.ds(s * RPT, RPT)])


def _scale0_tc(deg2, feat):
    def body(deg_ref, feat_ref, dis_ref, u_ref):
        d = deg_ref[0] + deg_ref[1]
        deg = d[:, :1]
        dis = lax.rsqrt(jnp.maximum(deg, 1.0))
        dis_ref[...] = dis
        u_ref[...] = feat_ref[...] * dis

    return pl.pallas_call(
        body,
        grid=(N // BR,),
        in_specs=[
            pl.BlockSpec((NC, BR, D), lambda i: (0, i, 0)),
            pl.BlockSpec((BR, D), lambda i: (i, 0)),
        ],
        out_specs=[
            pl.BlockSpec((BR, 1), lambda i: (i, 0)),
            pl.BlockSpec((BR, D), lambda i: (i, 0)),
        ],
        out_shape=[
            jax.ShapeDtypeStruct((N, 1), jnp.float32),
            jax.ShapeDtypeStruct((N, D), jnp.float32),
        ],
    )(deg2, feat)


def _advance_tc(f, s2, dis):
    def body(f_ref, s_ref, dis_ref, fn_ref, un_ref):
        ssum = s_ref[0] + s_ref[1]
        dis = dis_ref[...]
        fn = f_ref[...] + ssum * dis
        fn_ref[...] = fn
        un_ref[...] = fn * dis

    return pl.pallas_call(
        body,
        grid=(N // BR,),
        in_specs=[
            pl.BlockSpec((BR, D), lambda i: (i, 0)),
            pl.BlockSpec((NC, BR, D), lambda i: (0, i, 0)),
            pl.BlockSpec((BR, 1), lambda i: (i, 0)),
        ],
        out_specs=[
            pl.BlockSpec((BR, D), lambda i: (i, 0)),
            pl.BlockSpec((BR, D), lambda i: (i, 0)),
        ],
        out_shape=[
            jax.ShapeDtypeStruct((N, D), jnp.float32),
            jax.ShapeDtypeStruct((N, D), jnp.float32),
        ],
    )(f, s2, dis)


def _final_tc(f2, s2, dis, wpad):
    def body(f_ref, s_ref, dis_ref, w_ref, out_ref):
        ssum = s_ref[0] + s_ref[1]
        f2v = f_ref[...]
        g = f2v - ssum * dis_ref[...]
        a = jnp.maximum(w_ref[0:1, 0:1], 0.0) * 0.25
        b = (jnp.maximum(w_ref[0:1, 1:2], 0.0) * 0.5
             + jnp.maximum(w_ref[0:1, 2:3], 0.0) * 0.25)
        out_ref[...] = a * f2v + b * g

    return pl.pallas_call(
        body,
        grid=(N // BR,),
        in_specs=[
            pl.BlockSpec((BR, D), lambda i: (i, 0)),
            pl.BlockSpec((NC, BR, D), lambda i: (0, i, 0)),
            pl.BlockSpec((BR, 1), lambda i: (i, 0)),
            pl.BlockSpec((8, 4), lambda i: (0, 0)),
        ],
        out_specs=pl.BlockSpec((BR, D), lambda i: (i, 0)),
        out_shape=jax.ShapeDtypeStruct((N, D), jnp.float32),
    )(f2, s2, dis, wpad)


def kernel(feat, edge_index, weight):
    pad = EPAD - E
    src_p = jnp.concatenate([edge_index[0], jnp.zeros((pad,), jnp.int32)])
    dst_p = jnp.concatenate([edge_index[1], jnp.full((pad,), N, jnp.int32)])
    edges = jnp.stack([src_p, dst_p])
    wpad = jnp.pad(weight, (0, 29)).reshape(8, 4)

    deg2 = _deg_sc(edges)
    dis, u0 = _scale0_tc(deg2, feat)
    s0 = _prop_sc(u0, edges)
    f1, u1 = _advance_tc(feat, s0, dis)
    s1 = _prop_sc(u1, edges)
    f2, u2 = _advance_tc(f1, s1, dis)
    s2 = _prop_sc(u2, edges)
    return _final_tc(f2, s2, dis, wpad)
